# Initial kernel scaffold; baseline (speedup 1.0000x reference)
import jax
import jax.numpy as jnp
from jax.experimental import pallas as pl


def kernel(pixel_values, fs, proj_w, proj_b, pos_emb, cls_token, fs_w0, fs_b0, fs_w2, fs_b2):
    raise NotImplementedError("write your pallas kernel here")


# speedup vs baseline: 2.3687x; 2.3687x over previous
import math

import jax
import jax.numpy as jnp
from jax.experimental import pallas as pl
from jax.experimental.pallas import tpu as pltpu

B, L, PATCH_DIM, H = 8, 1024, 768, 1024
FREQ = 256
HALF = FREQ // 2
_LOG_MAX_PERIOD = math.log(10000.0)


def _body(fs_ref, px_ref, w_ref, pos_ref, cls_ref, w0_ref, b0_ref,
          w2_ref, b2_ref, out_ref, tok_ref):
    b = pl.program_id(0)

    @pl.when(b == 0)
    def _fs_tokens():
        k = jax.lax.broadcasted_iota(jnp.int32, (1, HALF), 1).astype(jnp.float32)
        freqs = jnp.exp((-_LOG_MAX_PERIOD / HALF) * k)
        args = fs_ref[...] * freqs
        emb = jnp.concatenate([jnp.cos(args), jnp.sin(args)], axis=-1)
        t = jax.lax.dot_general(
            emb, w0_ref[...], (((1,), (1,)), ((), ())),
            preferred_element_type=jnp.float32) + b0_ref[...]
        t = t * jax.nn.sigmoid(t)
        tok_ref[...] = jax.lax.dot_general(
            t, w2_ref[...], (((1,), (1,)), ((), ())),
            preferred_element_type=jnp.float32) + b2_ref[...]

    px = px_ref[0].astype(jnp.bfloat16)
    x = jax.lax.dot_general(
        px, w_ref[...], (((1,), (1,)), ((), ())),
        preferred_element_type=jnp.float32)
    out_ref[0, 2:, :] = x + pos_ref[...]
    out_ref[0, pl.ds(1, 1), :] = tok_ref[pl.ds(b, 1), :]
    out_ref[0, pl.ds(0, 1), :] = cls_ref[...]


def kernel(pixel_values, fs, proj_w, proj_b, pos_emb, cls_token,
           fs_w0, fs_b0, fs_w2, fs_b2):
    w_bf = proj_w.astype(jnp.bfloat16)
    pos_pb = pos_emb[:L] + proj_b[None, :]
    cls2 = cls_token.reshape(1, H)
    fs2 = fs.reshape(B, 1)
    b0 = fs_b0.reshape(1, H)
    b2 = fs_b2.reshape(1, H)

    const = lambda *_: (0, 0)
    out = pl.pallas_call(
        _body,
        grid=(B,),
        in_specs=[
            pl.BlockSpec((B, 1), const),
            pl.BlockSpec((1, L, PATCH_DIM), lambda b: (b, 0, 0)),
            pl.BlockSpec((H, PATCH_DIM), const),
            pl.BlockSpec((L, H), const),
            pl.BlockSpec((1, H), const),
            pl.BlockSpec((H, FREQ), const),
            pl.BlockSpec((1, H), const),
            pl.BlockSpec((H, H), const),
            pl.BlockSpec((1, H), const),
        ],
        out_specs=pl.BlockSpec((1, L + 2, H), lambda b: (b, 0, 0)),
        out_shape=jax.ShapeDtypeStruct((B, L + 2, H), jnp.float32),
        scratch_shapes=[pltpu.VMEM((B, H), jnp.float32)],
        compiler_params=pltpu.CompilerParams(
            dimension_semantics=("arbitrary",)),
    )(fs2, pixel_values, w_bf, pos_pb, cls2, fs_w0, b0, fs_w2, b2)
    return out
